# Initial kernel scaffold; baseline (speedup 1.0000x reference)
import functools

import jax
import jax.numpy as jnp
from jax import lax
from jax.experimental import pallas as pl
from jax.experimental.pallas import tpu as pltpu
from jax.experimental.pallas import tpu_sc as plsc

_N = 10000
_E = 320000
_DIN = 128
_DH = 16
_DOUT = 2
_NPAD = 10240

_NC = 2
_NS = 16
_NW = _NC * _NS
_EPT = _E // _NW
_CHUNK = 100
_NCHUNK = _EPT // _CHUNK
_ZROWS = 128
_RPT = _NPAD // _NS


def _sc_mesh():
    return plsc.VectorSubcoreMesh(core_axis_name="c", subcore_axis_name="s")


def _zero_shared(fill_v, acc_sh, s):
    def zrow(i, carry):
        fill_v[i, :] = jnp.zeros((_DH,), jnp.float32)
        return carry
    lax.fori_loop(0, _ZROWS, zrow, 0)
    row0 = s * _RPT
    for b in range(_RPT // _ZROWS):
        pltpu.sync_copy(fill_v, acc_sh.at[pl.ds(row0 + b * _ZROWS, _ZROWS)])


def _writeout_shared(acc_sh, out_hbm, c, s):
    row0 = s * _RPT
    pltpu.sync_copy(acc_sh.at[pl.ds(row0, _RPT)],
                    out_hbm.at[c].at[pl.ds(row0, _RPT)])


def _sc_degree(dstw):
    @functools.partial(
        pl.kernel,
        mesh=_sc_mesh(),
        out_type=jax.ShapeDtypeStruct((_NC, _NPAD, _DH), jnp.float32),
        scratch_types=[
            pltpu.VMEM((_NCHUNK, _CHUNK), jnp.int32),
            pltpu.VMEM((_ZROWS, _DH), jnp.float32),
            pltpu.VMEM_SHARED((_NPAD, _DH), jnp.float32),
        ],
    )
    def k(dst_hbm, out_hbm, dst_v, fill_v, acc_sh):
        c = lax.axis_index("c")
        s = lax.axis_index("s")
        wid = c * _NS + s
        pltpu.sync_copy(dst_hbm.at[wid], dst_v)
        _zero_shared(fill_v, acc_sh, s)
        plsc.subcore_barrier()

        def orow(i, carry):
            fill_v[i, :] = jnp.ones((_DH,), jnp.float32)
            return carry
        lax.fori_loop(0, _CHUNK, orow, 0)

        def body(j, carry):
            pltpu.sync_copy(fill_v.at[pl.ds(0, _CHUNK)],
                            acc_sh.at[dst_v.at[j]], add=True)
            return carry
        lax.fori_loop(0, _NCHUNK, body, 0)
        plsc.subcore_barrier()
        _writeout_shared(acc_sh, out_hbm, c, s)

    return k(dstw)


def _sc_gather_scatter_add(table, srcw, dstw):
    @functools.partial(
        pl.kernel,
        mesh=_sc_mesh(),
        out_type=jax.ShapeDtypeStruct((_NC, _NPAD, _DH), jnp.float32),
        scratch_types=[
            pltpu.VMEM((_NCHUNK, _CHUNK), jnp.int32),
            pltpu.VMEM((_NCHUNK, _CHUNK), jnp.int32),
            pltpu.VMEM((_CHUNK, _DH), jnp.float32),
            pltpu.VMEM((_ZROWS, _DH), jnp.float32),
            pltpu.VMEM_SHARED((_NPAD, _DH), jnp.float32),
            pltpu.SemaphoreType.DMA,
        ],
    )
    def k(tab_hbm, src_hbm, dst_hbm, out_hbm,
          src_v, dst_v, rows_v, fill_v, acc_sh, sem):
        c = lax.axis_index("c")
        s = lax.axis_index("s")
        wid = c * _NS + s
        pltpu.sync_copy(src_hbm.at[wid], src_v)
        pltpu.sync_copy(dst_hbm.at[wid], dst_v)
        _zero_shared(fill_v, acc_sh, s)
        plsc.subcore_barrier()

        def body(j, carry):
            pltpu.async_copy(tab_hbm.at[src_v.at[j]], rows_v, sem).wait()
            pltpu.sync_copy(rows_v, acc_sh.at[dst_v.at[j]], add=True)
            return carry
        lax.fori_loop(0, _NCHUNK, body, 0)
        plsc.subcore_barrier()
        _writeout_shared(acc_sh, out_hbm, c, s)

    return k(table, srcw, dstw)


def _tc_first(x_pad, W1, degparts):
    def body(x_ref, w_ref, dp_ref, hs_ref, dis_ref):
        deg = dp_ref[0][:, 0:1] + dp_ref[1][:, 0:1] + 1.0
        dis = lax.rsqrt(deg)
        p = jnp.dot(x_ref[...], w_ref[...], preferred_element_type=jnp.float32)
        hs_ref[...] = p * dis
        dis_ref[...] = dis

    return pl.pallas_call(
        body,
        out_shape=(jax.ShapeDtypeStruct((_NPAD, _DH), jnp.float32),
                   jax.ShapeDtypeStruct((_NPAD, 1), jnp.float32)),
    )(x_pad, W1, degparts)


def _tc_mid(aggparts, hs, dis, b, g, be, W_next):
    def body(ap_ref, hs_ref, dis_ref, b_ref, g_ref, be_ref, w_ref, out_ref):
        dis = dis_ref[...]
        agg = ap_ref[0] + ap_ref[1] + hs_ref[...]
        conv = agg * dis + b_ref[...]
        r = jnp.maximum(conv, 0.0)
        rv = r[:_N, :]
        m = jnp.mean(rv, axis=0, keepdims=True)
        v = jnp.mean((rv - m) * (rv - m), axis=0, keepdims=True)
        hn = (r - m) * lax.rsqrt(v + 1e-5) * g_ref[...] + be_ref[...]
        p = jnp.dot(hn, w_ref[...], preferred_element_type=jnp.float32)
        out_ref[...] = p * dis

    return pl.pallas_call(
        body,
        out_shape=jax.ShapeDtypeStruct((_NPAD, _DH), jnp.float32),
    )(aggparts, hs, dis, b, g, be, W_next)


def _tc_final(aggparts, hs, dis, b, fcW, fcb):
    def body(ap_ref, hs_ref, dis_ref, b_ref, w_ref, fb_ref, out_ref):
        agg = ap_ref[0] + ap_ref[1] + hs_ref[...]
        conv = agg * dis_ref[...] + b_ref[...]
        logits = jnp.dot(conv, w_ref[...],
                         preferred_element_type=jnp.float32) + fb_ref[...]
        mx = jnp.max(logits, axis=1, keepdims=True)
        e = jnp.exp(logits - mx)
        lse = mx + jnp.log(jnp.sum(e, axis=1, keepdims=True))
        out_ref[...] = logits - lse

    return pl.pallas_call(
        body,
        out_shape=jax.ShapeDtypeStruct((_NPAD, _DOUT), jnp.float32),
    )(aggparts, hs, dis, b, fcW, fcb)


def kernel(x, edge_index, W1, b1, W2, b2, W3, b3, g1, be1, g2, be2, fcW, fcb):
    src = edge_index[0].reshape(_NW, _NCHUNK, _CHUNK)
    dst = edge_index[1].reshape(_NW, _NCHUNK, _CHUNK)
    x_pad = jnp.pad(x, ((0, _NPAD - _N), (0, 0)))

    degparts = _sc_degree(dst)
    hs1, dis = _tc_first(x_pad, W1, degparts)
    agg1 = _sc_gather_scatter_add(hs1, src, dst)
    hs2 = _tc_mid(agg1, hs1, dis, b1.reshape(1, -1), g1.reshape(1, -1),
                  be1.reshape(1, -1), W2)
    agg2 = _sc_gather_scatter_add(hs2, src, dst)
    hs3 = _tc_mid(agg2, hs2, dis, b2.reshape(1, -1), g2.reshape(1, -1),
                  be2.reshape(1, -1), W3)
    agg3 = _sc_gather_scatter_add(hs3, src, dst)
    out = _tc_final(agg3, hs3, dis, b3.reshape(1, -1), fcW, fcb.reshape(1, -1))
    return out[:_N]


# speedup vs baseline: 27.8066x; 27.8066x over previous
import functools

import jax
import jax.numpy as jnp
from jax import lax
from jax.experimental import pallas as pl
from jax.experimental.pallas import tpu as pltpu
from jax.experimental.pallas import tpu_sc as plsc

_N = 10000
_E = 320000
_DIN = 128
_DH = 16
_DOUT = 2
_NPAD = 10240

_NC = 2
_NS = 16
_NW = _NC * _NS
_EPT = _E // _NW
_CHUNK = 100
_NCHUNK = _EPT // _CHUNK
_ZROWS = 128
_RPT = _NPAD // _NS


def _sc_mesh():
    return plsc.VectorSubcoreMesh(core_axis_name="c", subcore_axis_name="s")


_SC_PARAMS = pltpu.CompilerParams(use_tc_tiling_on_sc=False)


def _zero_shared(fill_v, acc_sh, s):
    def zrow(i, carry):
        fill_v[i, :] = jnp.zeros((_DH,), jnp.float32)
        return carry
    lax.fori_loop(0, _ZROWS, zrow, 0)
    row0 = s * _RPT
    for b in range(_RPT // _ZROWS):
        pltpu.sync_copy(fill_v, acc_sh.at[pl.ds(row0 + b * _ZROWS, _ZROWS)])


def _writeout_shared(acc_sh, out_hbm, c, s):
    row0 = s * _RPT
    pltpu.sync_copy(acc_sh.at[pl.ds(row0, _RPT)],
                    out_hbm.at[c].at[pl.ds(row0, _RPT)])


def _sc_degree(dstw):
    @functools.partial(
        pl.kernel,
        mesh=_sc_mesh(),
        out_type=jax.ShapeDtypeStruct((_NC, _NPAD, _DH), jnp.float32),
        compiler_params=_SC_PARAMS,
        scratch_types=[
            pltpu.VMEM((_NCHUNK, _CHUNK), jnp.int32),
            pltpu.VMEM((_ZROWS, _DH), jnp.float32),
            pltpu.VMEM_SHARED((_NPAD, _DH), jnp.float32),
        ],
    )
    def k(dst_hbm, out_hbm, dst_v, fill_v, acc_sh):
        c = lax.axis_index("c")
        s = lax.axis_index("s")
        wid = c * _NS + s
        pltpu.sync_copy(dst_hbm.at[wid], dst_v)
        _zero_shared(fill_v, acc_sh, s)
        plsc.subcore_barrier()

        def orow(i, carry):
            fill_v[i, :] = jnp.ones((_DH,), jnp.float32)
            return carry
        lax.fori_loop(0, _CHUNK, orow, 0)

        def body(j, carry):
            pltpu.sync_copy(fill_v.at[pl.ds(0, _CHUNK)],
                            acc_sh.at[dst_v.at[j]], add=True)
            return carry
        lax.fori_loop(0, _NCHUNK, body, 0)
        plsc.subcore_barrier()
        _writeout_shared(acc_sh, out_hbm, c, s)

    return k(dstw)


def _sc_gather_scatter_add(table, srcw, dstw):
    @functools.partial(
        pl.kernel,
        mesh=_sc_mesh(),
        out_type=jax.ShapeDtypeStruct((_NC, _NPAD, _DH), jnp.float32),
        compiler_params=_SC_PARAMS,
        scratch_types=[
            pltpu.VMEM((_NCHUNK, _CHUNK), jnp.int32),
            pltpu.VMEM((_NCHUNK, _CHUNK), jnp.int32),
            pltpu.VMEM((_CHUNK, _DH), jnp.float32),
            pltpu.VMEM((_ZROWS, _DH), jnp.float32),
            pltpu.VMEM_SHARED((_NPAD, _DH), jnp.float32),
            pltpu.SemaphoreType.DMA,
        ],
    )
    def k(tab_hbm, src_hbm, dst_hbm, out_hbm,
          src_v, dst_v, rows_v, fill_v, acc_sh, sem):
        c = lax.axis_index("c")
        s = lax.axis_index("s")
        wid = c * _NS + s
        pltpu.sync_copy(src_hbm.at[wid], src_v)
        pltpu.sync_copy(dst_hbm.at[wid], dst_v)
        _zero_shared(fill_v, acc_sh, s)
        plsc.subcore_barrier()

        def body(j, carry):
            pltpu.async_copy(tab_hbm.at[src_v.at[j]], rows_v, sem).wait()
            pltpu.sync_copy(rows_v, acc_sh.at[dst_v.at[j]], add=True)
            return carry
        lax.fori_loop(0, _NCHUNK, body, 0)
        plsc.subcore_barrier()
        _writeout_shared(acc_sh, out_hbm, c, s)

    return k(table, srcw, dstw)


def _tc_first(x_pad, W1, degparts):
    def body(x_ref, w_ref, dp_ref, hs_ref, dis_ref):
        deg = dp_ref[0][:, 0:1] + dp_ref[1][:, 0:1] + 1.0
        dis = lax.rsqrt(deg)
        p = jnp.dot(x_ref[...], w_ref[...], preferred_element_type=jnp.float32)
        hs_ref[...] = p * dis
        dis_ref[...] = dis

    return pl.pallas_call(
        body,
        out_shape=(jax.ShapeDtypeStruct((_NPAD, _DH), jnp.float32),
                   jax.ShapeDtypeStruct((_NPAD, 1), jnp.float32)),
    )(x_pad, W1, degparts)


def _tc_mid(aggparts, hs, dis, b, g, be, W_next):
    def body(ap_ref, hs_ref, dis_ref, b_ref, g_ref, be_ref, w_ref, out_ref):
        dis = dis_ref[...]
        agg = ap_ref[0] + ap_ref[1] + hs_ref[...]
        conv = agg * dis + b_ref[...]
        r = jnp.maximum(conv, 0.0)
        rv = r[:_N, :]
        m = jnp.mean(rv, axis=0, keepdims=True)
        v = jnp.mean((rv - m) * (rv - m), axis=0, keepdims=True)
        hn = (r - m) * lax.rsqrt(v + 1e-5) * g_ref[...] + be_ref[...]
        p = jnp.dot(hn, w_ref[...], preferred_element_type=jnp.float32)
        out_ref[...] = p * dis

    return pl.pallas_call(
        body,
        out_shape=jax.ShapeDtypeStruct((_NPAD, _DH), jnp.float32),
    )(aggparts, hs, dis, b, g, be, W_next)


def _tc_final(aggparts, hs, dis, b, fcW, fcb):
    def body(ap_ref, hs_ref, dis_ref, b_ref, w_ref, fb_ref, out_ref):
        agg = ap_ref[0] + ap_ref[1] + hs_ref[...]
        conv = agg * dis_ref[...] + b_ref[...]
        logits = jnp.dot(conv, w_ref[...],
                         preferred_element_type=jnp.float32) + fb_ref[...]
        mx = jnp.max(logits, axis=1, keepdims=True)
        e = jnp.exp(logits - mx)
        lse = mx + jnp.log(jnp.sum(e, axis=1, keepdims=True))
        out_ref[...] = logits - lse

    return pl.pallas_call(
        body,
        out_shape=jax.ShapeDtypeStruct((_NPAD, _DOUT), jnp.float32),
    )(aggparts, hs, dis, b, fcW, fcb)


def kernel(x, edge_index, W1, b1, W2, b2, W3, b3, g1, be1, g2, be2, fcW, fcb):
    src = edge_index[0].reshape(_NW, _NCHUNK, _CHUNK)
    dst = edge_index[1].reshape(_NW, _NCHUNK, _CHUNK)
    x_pad = jnp.pad(x, ((0, _NPAD - _N), (0, 0)))

    degparts = _sc_degree(dst)
    hs1, dis = _tc_first(x_pad, W1, degparts)
    agg1 = _sc_gather_scatter_add(hs1, src, dst)
    hs2 = _tc_mid(agg1, hs1, dis, b1.reshape(1, -1), g1.reshape(1, -1),
                  be1.reshape(1, -1), W2)
    agg2 = _sc_gather_scatter_add(hs2, src, dst)
    hs3 = _tc_mid(agg2, hs2, dis, b2.reshape(1, -1), g2.reshape(1, -1),
                  be2.reshape(1, -1), W3)
    agg3 = _sc_gather_scatter_add(hs3, src, dst)
    out = _tc_final(agg3, hs3, dis, b3.reshape(1, -1), fcW, fcb.reshape(1, -1))
    return out[:_N]


# speedup vs baseline: 55.1263x; 1.9825x over previous
import functools

import jax
import jax.numpy as jnp
from jax import lax
from jax.experimental import pallas as pl
from jax.experimental.pallas import tpu as pltpu
from jax.experimental.pallas import tpu_sc as plsc

_N = 10000
_E = 320000
_DIN = 128
_DH = 16
_DOUT = 2
_NPAD = 10240

_NC = 2
_NS = 16
_NW = _NC * _NS
_EPT = _E // _NW
_CHUNK = 125
_NCHUNK = _EPT // _CHUNK
_KDEP = 8
_NGRP = _NCHUNK // _KDEP
_ZROWS = 128
_RPT = _NPAD // _NS


def _sc_mesh():
    return plsc.VectorSubcoreMesh(core_axis_name="c", subcore_axis_name="s")


_SC_PARAMS = pltpu.CompilerParams(use_tc_tiling_on_sc=False)


def _zero_shared(fill_v, acc_sh, s):
    def zrow(i, carry):
        fill_v[i, :] = jnp.zeros((_DH,), jnp.float32)
        return carry
    lax.fori_loop(0, _ZROWS, zrow, 0)
    row0 = s * _RPT
    for b in range(_RPT // _ZROWS):
        pltpu.sync_copy(fill_v, acc_sh.at[pl.ds(row0 + b * _ZROWS, _ZROWS)])


def _writeout_shared(acc_sh, out_hbm, c, s):
    row0 = s * _RPT
    pltpu.sync_copy(acc_sh.at[pl.ds(row0, _RPT)],
                    out_hbm.at[c].at[pl.ds(row0, _RPT)])


def _sc_degree(dstw):
    @functools.partial(
        pl.kernel,
        mesh=_sc_mesh(),
        out_type=jax.ShapeDtypeStruct((_NC, _NPAD, _DH), jnp.float32),
        compiler_params=_SC_PARAMS,
        scratch_types=[
            pltpu.VMEM((_NCHUNK, _CHUNK), jnp.int32),
            pltpu.VMEM((_ZROWS, _DH), jnp.float32),
            pltpu.VMEM_SHARED((_NPAD, _DH), jnp.float32),
        ],
    )
    def k(dst_hbm, out_hbm, dst_v, fill_v, acc_sh):
        c = lax.axis_index("c")
        s = lax.axis_index("s")
        wid = c * _NS + s
        pltpu.sync_copy(dst_hbm.at[wid], dst_v)
        _zero_shared(fill_v, acc_sh, s)
        plsc.subcore_barrier()

        def orow(i, carry):
            fill_v[i, :] = jnp.ones((_DH,), jnp.float32)
            return carry
        lax.fori_loop(0, _CHUNK, orow, 0)

        def body(j, carry):
            pltpu.sync_copy(fill_v.at[pl.ds(0, _CHUNK)],
                            acc_sh.at[dst_v.at[j]], add=True)
            return carry
        lax.fori_loop(0, _NCHUNK, body, 0)
        plsc.subcore_barrier()
        _writeout_shared(acc_sh, out_hbm, c, s)

    return k(dstw)


def _sc_gather_scatter_add(table, srcw, dstw):
    @functools.partial(
        pl.kernel,
        mesh=_sc_mesh(),
        out_type=jax.ShapeDtypeStruct((_NC, _NPAD, _DH), jnp.float32),
        compiler_params=_SC_PARAMS,
        scratch_types=[
            pltpu.VMEM((_NCHUNK, _CHUNK), jnp.int32),
            pltpu.VMEM((_NCHUNK, _CHUNK), jnp.int32),
            pltpu.VMEM((_KDEP, _CHUNK, _DH), jnp.float32),
            pltpu.VMEM((_ZROWS, _DH), jnp.float32),
            pltpu.VMEM_SHARED((_NPAD, _DH), jnp.float32),
        ] + [pltpu.SemaphoreType.DMA] * _KDEP,
    )
    def k(tab_hbm, src_hbm, dst_hbm, out_hbm,
          src_v, dst_v, rows_v, fill_v, acc_sh, *sems):
        c = lax.axis_index("c")
        s = lax.axis_index("s")
        wid = c * _NS + s
        pltpu.sync_copy(src_hbm.at[wid], src_v)
        pltpu.sync_copy(dst_hbm.at[wid], dst_v)
        _zero_shared(fill_v, acc_sh, s)
        plsc.subcore_barrier()

        for b in range(_KDEP):
            pltpu.async_copy(tab_hbm.at[src_v.at[b]], rows_v.at[b], sems[b])

        def body(j, carry):
            for b in range(_KDEP):
                ch = j * _KDEP + b
                pltpu.make_async_copy(
                    tab_hbm.at[src_v.at[ch]], rows_v.at[b], sems[b]).wait()
                pltpu.sync_copy(rows_v.at[b], acc_sh.at[dst_v.at[ch]],
                                add=True)

                @pl.when(j < _NGRP - 1)
                def _refill():
                    pltpu.async_copy(tab_hbm.at[src_v.at[ch + _KDEP]],
                                     rows_v.at[b], sems[b])
            return carry
        lax.fori_loop(0, _NGRP, body, 0)
        plsc.subcore_barrier()
        _writeout_shared(acc_sh, out_hbm, c, s)

    return k(table, srcw, dstw)


def _tc_first(x_pad, W1, degparts):
    def body(x_ref, w_ref, dp_ref, hs_ref, dis_ref):
        deg = dp_ref[0][:, 0:1] + dp_ref[1][:, 0:1] + 1.0
        dis = lax.rsqrt(deg)
        p = jnp.dot(x_ref[...], w_ref[...], preferred_element_type=jnp.float32)
        hs_ref[...] = p * dis
        dis_ref[...] = dis

    return pl.pallas_call(
        body,
        out_shape=(jax.ShapeDtypeStruct((_NPAD, _DH), jnp.float32),
                   jax.ShapeDtypeStruct((_NPAD, 1), jnp.float32)),
    )(x_pad, W1, degparts)


def _tc_mid(aggparts, hs, dis, b, g, be, W_next):
    def body(ap_ref, hs_ref, dis_ref, b_ref, g_ref, be_ref, w_ref, out_ref):
        dis = dis_ref[...]
        agg = ap_ref[0] + ap_ref[1] + hs_ref[...]
        conv = agg * dis + b_ref[...]
        r = jnp.maximum(conv, 0.0)
        rv = r[:_N, :]
        m = jnp.mean(rv, axis=0, keepdims=True)
        v = jnp.mean((rv - m) * (rv - m), axis=0, keepdims=True)
        hn = (r - m) * lax.rsqrt(v + 1e-5) * g_ref[...] + be_ref[...]
        p = jnp.dot(hn, w_ref[...], preferred_element_type=jnp.float32)
        out_ref[...] = p * dis

    return pl.pallas_call(
        body,
        out_shape=jax.ShapeDtypeStruct((_NPAD, _DH), jnp.float32),
    )(aggparts, hs, dis, b, g, be, W_next)


def _tc_final(aggparts, hs, dis, b, fcW, fcb):
    def body(ap_ref, hs_ref, dis_ref, b_ref, w_ref, fb_ref, out_ref):
        agg = ap_ref[0] + ap_ref[1] + hs_ref[...]
        conv = agg * dis_ref[...] + b_ref[...]
        logits = jnp.dot(conv, w_ref[...],
                         preferred_element_type=jnp.float32) + fb_ref[...]
        mx = jnp.max(logits, axis=1, keepdims=True)
        e = jnp.exp(logits - mx)
        lse = mx + jnp.log(jnp.sum(e, axis=1, keepdims=True))
        out_ref[...] = logits - lse

    return pl.pallas_call(
        body,
        out_shape=jax.ShapeDtypeStruct((_NPAD, _DOUT), jnp.float32),
    )(aggparts, hs, dis, b, fcW, fcb)


def kernel(x, edge_index, W1, b1, W2, b2, W3, b3, g1, be1, g2, be2, fcW, fcb):
    src = edge_index[0].reshape(_NW, _NCHUNK, _CHUNK)
    dst = edge_index[1].reshape(_NW, _NCHUNK, _CHUNK)
    x_pad = jnp.pad(x, ((0, _NPAD - _N), (0, 0)))

    degparts = _sc_degree(dst)
    hs1, dis = _tc_first(x_pad, W1, degparts)
    agg1 = _sc_gather_scatter_add(hs1, src, dst)
    hs2 = _tc_mid(agg1, hs1, dis, b1.reshape(1, -1), g1.reshape(1, -1),
                  be1.reshape(1, -1), W2)
    agg2 = _sc_gather_scatter_add(hs2, src, dst)
    hs3 = _tc_mid(agg2, hs2, dis, b2.reshape(1, -1), g2.reshape(1, -1),
                  be2.reshape(1, -1), W3)
    agg3 = _sc_gather_scatter_add(hs3, src, dst)
    out = _tc_final(agg3, hs3, dis, b3.reshape(1, -1), fcW, fcb.reshape(1, -1))
    return out[:_N]
